# baseline (speedup 1.0000x reference)
import jax
import jax.numpy as jnp
from jax import lax
from jax.experimental import pallas as pl
from jax.experimental.pallas import tpu as pltpu
from jax.experimental.pallas import tpu_sc as plsc

N_TOKENS = 32768
N_EXPERTS = 64
K = 8
L = 16
NC = 2
NS = 16
NW = NC * NS
TOK_PER_W = N_TOKENS // NW
GROUPS = TOK_PER_W // L

IDMASK = (1 << 6) - 1

SORT8 = [(0, 1), (2, 3), (4, 5), (6, 7),
         (0, 2), (1, 3), (4, 6), (5, 7),
         (1, 2), (5, 6),
         (0, 4), (1, 5), (2, 6), (3, 7),
         (2, 4), (3, 5),
         (1, 2), (3, 4), (5, 6)]

BITONIC8 = [(0, 4), (1, 5), (2, 6), (3, 7),
            (0, 2), (1, 3), (4, 6), (5, 7),
            (0, 1), (2, 3), (4, 5), (6, 7)]


def _sort8_desc(v):
    for i, j in SORT8:
        hi = jnp.maximum(v[i], v[j])
        lo = jnp.minimum(v[i], v[j])
        v[i], v[j] = hi, lo
    return v


def _merge_top8(a, b):
    m = [jnp.maximum(a[i], b[7 - i]) for i in range(8)]
    for i, j in BITONIC8:
        hi = jnp.maximum(m[i], m[j])
        lo = jnp.minimum(m[i], m[j])
        m[i], m[j] = hi, lo
    return m


HALF = TOK_PER_W // 2


def _topk_body(lt_hbm, w_hbm, ids_hbm, lt_out_hbm, logits_v, w_v, ids_v,
               sem_in, sem_out):
    wid = lax.axis_index("s") * NC + lax.axis_index("c")
    t0 = wid * TOK_PER_W
    pltpu.sync_copy(lt_hbm.at[:, pl.ds(t0, HALF)],
                    logits_v.at[:, pl.ds(0, HALF)])
    fetch2 = pltpu.make_async_copy(
        lt_hbm.at[:, pl.ds(t0 + HALF, HALF)],
        logits_v.at[:, pl.ds(HALF, HALF)], sem_in)
    fetch2.start()
    pass1 = pltpu.make_async_copy(
        logits_v.at[:, pl.ds(0, HALF)],
        lt_out_hbm.at[:, pl.ds(t0, HALF)], sem_out)
    pass1.start()

    def group(i):
        off = i * L
        packed = []
        for e in range(N_EXPERTS):
            v = logits_v[e, pl.ds(off, L)]
            vi = plsc.bitcast(v, jnp.int32)
            vi = (vi & ~IDMASK) | (IDMASK - e)
            packed.append(plsc.bitcast(vi, jnp.float32))
        tops = [_sort8_desc(packed[8 * g:8 * g + 8]) for g in range(8)]
        m01 = _merge_top8(tops[0], tops[1])
        m23 = _merge_top8(tops[2], tops[3])
        m45 = _merge_top8(tops[4], tops[5])
        m67 = _merge_top8(tops[6], tops[7])
        m03 = _merge_top8(m01, m23)
        m47 = _merge_top8(m45, m67)
        top = _merge_top8(m03, m47)

        es = [jnp.exp(top[j] - top[0]) for j in range(1, K)]
        denom = es[0]
        for e in es[1:]:
            denom = denom + e
        inv = 1.0 / (denom + 1.0)
        w_v[0, pl.ds(off, L)] = inv
        for j in range(1, K):
            w_v[j, pl.ds(off, L)] = es[j - 1] * inv
        for j in range(K):
            ti = plsc.bitcast(top[j], jnp.int32)
            ids_v[j, pl.ds(off, L)] = IDMASK - (ti & IDMASK)

    plsc.parallel_loop(0, GROUPS // 2, unroll=4)(group)
    fetch2.wait()
    pass2 = pltpu.make_async_copy(
        logits_v.at[:, pl.ds(HALF, HALF)],
        lt_out_hbm.at[:, pl.ds(t0 + HALF, HALF)], sem_out)
    pass2.start()
    plsc.parallel_loop(GROUPS // 2, GROUPS, unroll=4)(group)

    pltpu.sync_copy(w_v, w_hbm.at[:, pl.ds(t0, TOK_PER_W)])
    pltpu.sync_copy(ids_v, ids_hbm.at[:, pl.ds(t0, TOK_PER_W)])
    pass1.wait()
    pass2.wait()


def kernel(hidden_states, router_logits):
    del hidden_states
    fn = pl.kernel(
        _topk_body,
        out_type=(
            jax.ShapeDtypeStruct((K, N_TOKENS), jnp.float32),
            jax.ShapeDtypeStruct((K, N_TOKENS), jnp.int32),
            jax.ShapeDtypeStruct((N_EXPERTS, N_TOKENS), jnp.float32),
        ),
        mesh=plsc.VectorSubcoreMesh(core_axis_name="c", subcore_axis_name="s"),
        compiler_params=pltpu.CompilerParams(
            needs_layout_passes=False, skip_device_barrier=True),
        scratch_types=[
            pltpu.VMEM((N_EXPERTS, TOK_PER_W), jnp.float32),
            pltpu.VMEM((K, TOK_PER_W), jnp.float32),
            pltpu.VMEM((K, TOK_PER_W), jnp.int32),
            pltpu.SemaphoreType.DMA,
            pltpu.SemaphoreType.DMA,
        ],
    )
    w8, ids8, lt_out = fn(router_logits.T)
    return w8.T, ids8.T, lt_out.T


# speedup vs baseline: 1.0087x; 1.0087x over previous
import jax
import jax.numpy as jnp
from jax import lax
from jax.experimental import pallas as pl
from jax.experimental.pallas import tpu as pltpu
from jax.experimental.pallas import tpu_sc as plsc

N_TOKENS = 32768
N_EXPERTS = 64
K = 8
L = 16
NC = 2
NS = 16
NW = NC * NS
TOK_PER_W = N_TOKENS // NW
GROUPS = TOK_PER_W // L

IDMASK = (1 << 6) - 1

SORT8 = [(0, 1), (2, 3), (4, 5), (6, 7),
         (0, 2), (1, 3), (4, 6), (5, 7),
         (1, 2), (5, 6),
         (0, 4), (1, 5), (2, 6), (3, 7),
         (2, 4), (3, 5),
         (1, 2), (3, 4), (5, 6)]

BITONIC8 = [(0, 4), (1, 5), (2, 6), (3, 7),
            (0, 2), (1, 3), (4, 6), (5, 7),
            (0, 1), (2, 3), (4, 5), (6, 7)]


def _sort8_desc(v):
    for i, j in SORT8:
        hi = jnp.maximum(v[i], v[j])
        lo = jnp.minimum(v[i], v[j])
        v[i], v[j] = hi, lo
    return v


def _merge_top8(a, b):
    m = [jnp.maximum(a[i], b[7 - i]) for i in range(8)]
    for i, j in BITONIC8:
        hi = jnp.maximum(m[i], m[j])
        lo = jnp.minimum(m[i], m[j])
        m[i], m[j] = hi, lo
    return m


HALF = TOK_PER_W // 2


def _topk_body(lt_hbm, w_hbm, ids_hbm, lt_out_hbm, logits_v, w_v, ids_v,
               sem_in, sem_out):
    wid = lax.axis_index("s") * NC + lax.axis_index("c")
    t0 = wid * TOK_PER_W
    pltpu.sync_copy(lt_hbm.at[:, pl.ds(t0, HALF)],
                    logits_v.at[:, pl.ds(0, HALF)])
    fetch2 = pltpu.make_async_copy(
        lt_hbm.at[:, pl.ds(t0 + HALF, HALF)],
        logits_v.at[:, pl.ds(HALF, HALF)], sem_in)
    fetch2.start()
    pass1 = pltpu.make_async_copy(
        logits_v.at[:, pl.ds(0, HALF)],
        lt_out_hbm.at[:, pl.ds(t0, HALF)], sem_out)
    pass1.start()

    def group(i):
        off = i * L
        packed = []
        for e in range(N_EXPERTS):
            v = logits_v[e, pl.ds(off, L)]
            vi = plsc.bitcast(v, jnp.int32)
            vi = (vi & ~IDMASK) | (IDMASK - e)
            packed.append(plsc.bitcast(vi, jnp.float32))
        tops = [_sort8_desc(packed[8 * g:8 * g + 8]) for g in range(8)]
        m01 = _merge_top8(tops[0], tops[1])
        m23 = _merge_top8(tops[2], tops[3])
        m45 = _merge_top8(tops[4], tops[5])
        m67 = _merge_top8(tops[6], tops[7])
        m03 = _merge_top8(m01, m23)
        m47 = _merge_top8(m45, m67)
        top = _merge_top8(m03, m47)

        lanes = lax.iota(jnp.int32, L) + off
        ids = []
        ve = []
        for j in range(K):
            ti = plsc.bitcast(top[j], jnp.int32)
            idj = IDMASK - (ti & IDMASK)
            ids.append(idj)
            ve.append(plsc.load_gather(logits_v, [idj, lanes]))
        for i, j in [(0, 1), (2, 3), (4, 5), (6, 7),
                     (1, 2), (3, 4), (5, 6),
                     (0, 1), (2, 3), (4, 5), (6, 7)]:
            swap = ve[i] < ve[j]
            vi = jnp.where(swap, ve[j], ve[i])
            vj = jnp.where(swap, ve[i], ve[j])
            ve[i], ve[j] = vi, vj
            ii = jnp.where(swap, ids[j], ids[i])
            ij = jnp.where(swap, ids[i], ids[j])
            ids[i], ids[j] = ii, ij

        es = [jnp.exp(ve[j] - ve[0]) for j in range(1, K)]
        denom = es[0]
        for e in es[1:]:
            denom = denom + e
        inv = 1.0 / (denom + 1.0)
        w_v[0, pl.ds(off, L)] = inv
        for j in range(1, K):
            w_v[j, pl.ds(off, L)] = es[j - 1] * inv
        for j in range(K):
            ids_v[j, pl.ds(off, L)] = ids[j]

    plsc.parallel_loop(0, GROUPS // 2, unroll=2)(group)
    fetch2.wait()
    pass2 = pltpu.make_async_copy(
        logits_v.at[:, pl.ds(HALF, HALF)],
        lt_out_hbm.at[:, pl.ds(t0 + HALF, HALF)], sem_out)
    pass2.start()
    plsc.parallel_loop(GROUPS // 2, GROUPS, unroll=2)(group)

    pltpu.sync_copy(w_v, w_hbm.at[:, pl.ds(t0, TOK_PER_W)])
    pltpu.sync_copy(ids_v, ids_hbm.at[:, pl.ds(t0, TOK_PER_W)])
    pass1.wait()
    pass2.wait()


def kernel(hidden_states, router_logits):
    del hidden_states
    fn = pl.kernel(
        _topk_body,
        out_type=(
            jax.ShapeDtypeStruct((K, N_TOKENS), jnp.float32),
            jax.ShapeDtypeStruct((K, N_TOKENS), jnp.int32),
            jax.ShapeDtypeStruct((N_EXPERTS, N_TOKENS), jnp.float32),
        ),
        mesh=plsc.VectorSubcoreMesh(core_axis_name="c", subcore_axis_name="s"),
        compiler_params=pltpu.CompilerParams(
            needs_layout_passes=False, skip_device_barrier=True),
        scratch_types=[
            pltpu.VMEM((N_EXPERTS, TOK_PER_W), jnp.float32),
            pltpu.VMEM((K, TOK_PER_W), jnp.float32),
            pltpu.VMEM((K, TOK_PER_W), jnp.int32),
            pltpu.SemaphoreType.DMA,
            pltpu.SemaphoreType.DMA,
        ],
    )
    w8, ids8, lt_out = fn(router_logits.T)
    return w8.T, ids8.T, lt_out.T
